# baseline (speedup 1.0000x reference)
import functools

import jax
import jax.numpy as jnp
from jax.experimental import pallas as pl
from jax.experimental.pallas import tpu as pltpu


def _heads_kernel(oc, od, hb, w_ref, b_ref, x_ref,
                  oc_ref, or_ref, od_ref, wx_ref):
    cb = x_ref.shape[0]
    H = x_ref.shape[1]
    W = x_ref.shape[2]
    ot = b_ref.shape[0]
    n_g = wx_ref.shape[0]
    me = ot * hb
    ke = cb * hb
    b = pl.program_id(0)
    g = pl.program_id(1)
    dn = (((1,), (0,)), ((), ()))

    @pl.when(jnp.logical_and(b == 0, g == 0))
    def _expand_weights():
        rl = (jax.lax.broadcasted_iota(jnp.int32, (me, ot), 0) // hb
              == jax.lax.broadcasted_iota(jnp.int32, (me, ot), 1)
              ).astype(jnp.float32)
        rr = (jax.lax.broadcasted_iota(jnp.int32, (cb, ke), 1) // hb
              == jax.lax.broadcasted_iota(jnp.int32, (cb, ke), 0)
              ).astype(jnp.float32)
        mask = (jax.lax.broadcasted_iota(jnp.int32, (me, ke), 0) % hb
                == jax.lax.broadcasted_iota(jnp.int32, (me, ke), 1) % hb)
        for g2 in range(n_g):
            w_blk = w_ref[:, g2 * cb:(g2 + 1) * cb]
            wr = jax.lax.dot_general(
                rl, w_blk, dn, preferred_element_type=jnp.float32)
            wfull = jax.lax.dot_general(
                wr, rr, dn, preferred_element_type=jnp.float32)
            wx_ref[g2] = jnp.where(mask, wfull, 0.0)

    @pl.when(g == 0)
    def _init():
        bias = jnp.broadcast_to(b_ref[...][:, :, None], (ot, H, W))
        oc_ref[0] = bias[:oc]
        or_ref[0] = bias[oc:-od]
        od_ref[0] = bias[-od:]

    w_g = wx_ref[g]
    for t in range(H // hb):
        sl = slice(t * hb, (t + 1) * hb)
        x2d = x_ref[:, sl, :].reshape(ke, W)
        y = jax.lax.dot_general(
            w_g, x2d, dn, preferred_element_type=jnp.float32)
        y = y.reshape(ot, hb, W)
        oc_ref[0, :, sl, :] += y[:oc]
        or_ref[0, :, sl, :] += y[oc:-od]
        od_ref[0, :, sl, :] += y[-od:]


def kernel(x, W_cls, b_cls, W_reg, b_reg, W_dir, b_dir):
    B, C, H, W = x.shape
    Oc, Or, Od = W_cls.shape[0], W_reg.shape[0], W_dir.shape[0]
    Ot = Oc + Or + Od
    HB = 8
    CB = 64
    n_g = C // CB
    assert H % HB == 0 and C % CB == 0

    w_all = jnp.concatenate([W_cls, W_reg, W_dir], axis=0)
    b_all = jnp.concatenate([b_cls, b_reg, b_dir], axis=0)[:, None]
    x3 = x.reshape(B * C, H, W)

    body = functools.partial(_heads_kernel, Oc, Od, HB)
    outs = pl.pallas_call(
        body,
        grid=(B, n_g),
        in_specs=[
            pl.BlockSpec((Ot, C), lambda b, g: (0, 0)),
            pl.BlockSpec((Ot, 1), lambda b, g: (0, 0)),
            pl.BlockSpec((CB, H, W), lambda b, g: (b * n_g + g, 0, 0)),
        ],
        out_specs=[
            pl.BlockSpec((1, Oc, H, W), lambda b, g: (b, 0, 0, 0)),
            pl.BlockSpec((1, Or, H, W), lambda b, g: (b, 0, 0, 0)),
            pl.BlockSpec((1, Od, H, W), lambda b, g: (b, 0, 0, 0)),
        ],
        out_shape=[
            jax.ShapeDtypeStruct((B, Oc, H, W), jnp.float32),
            jax.ShapeDtypeStruct((B, Or, H, W), jnp.float32),
            jax.ShapeDtypeStruct((B, Od, H, W), jnp.float32),
        ],
        scratch_shapes=[
            pltpu.VMEM((n_g, Ot * HB, CB * HB), jnp.float32),
        ],
        compiler_params=pltpu.CompilerParams(
            dimension_semantics=("parallel", "parallel")),
    )(w_all, b_all, x3)
    return tuple(outs)


# speedup vs baseline: 1.0036x; 1.0036x over previous
import functools

import jax
import jax.numpy as jnp
from jax.experimental import pallas as pl
from jax.experimental.pallas import tpu as pltpu


def _heads_kernel(oc, od, hb, w_ref, b_ref, x_ref,
                  oc_ref, or_ref, od_ref, wx_ref):
    cb = x_ref.shape[0]
    H = x_ref.shape[1]
    W = x_ref.shape[2]
    ot = b_ref.shape[0]
    n_g = wx_ref.shape[0]
    me = ot * hb
    ke = cb * hb
    b = pl.program_id(0)
    g = pl.program_id(1)
    dn = (((1,), (0,)), ((), ()))

    @pl.when(jnp.logical_and(b == 0, g == 0))
    def _expand_weights():
        rl = (jax.lax.broadcasted_iota(jnp.int32, (me, ot), 0) // hb
              == jax.lax.broadcasted_iota(jnp.int32, (me, ot), 1)
              ).astype(jnp.float32)
        rr = (jax.lax.broadcasted_iota(jnp.int32, (cb, ke), 1) // hb
              == jax.lax.broadcasted_iota(jnp.int32, (cb, ke), 0)
              ).astype(jnp.float32)
        mask = (jax.lax.broadcasted_iota(jnp.int32, (me, ke), 0) % hb
                == jax.lax.broadcasted_iota(jnp.int32, (me, ke), 1) % hb)
        for g2 in range(n_g):
            w_blk = w_ref[:, g2 * cb:(g2 + 1) * cb]
            wr = jax.lax.dot_general(
                rl, w_blk, dn, preferred_element_type=jnp.float32)
            wfull = jax.lax.dot_general(
                wr, rr, dn, preferred_element_type=jnp.float32)
            wx_ref[g2] = jnp.where(mask, wfull, 0.0)

    @pl.when(g == 0)
    def _init():
        bias = jnp.broadcast_to(b_ref[...][:, :, None], (ot, H, W))
        oc_ref[0] = bias[:oc]
        or_ref[0] = bias[oc:-od]
        od_ref[0] = bias[-od:]

    w_g = wx_ref[g]
    for t in range(H // hb):
        sl = slice(t * hb, (t + 1) * hb)
        x2d = x_ref[:, sl, :].reshape(ke, W)
        y = jax.lax.dot_general(
            w_g, x2d, dn, preferred_element_type=jnp.float32)
        y = y.reshape(ot, hb, W)
        oc_ref[0, :, sl, :] += y[:oc]
        or_ref[0, :, sl, :] += y[oc:-od]
        od_ref[0, :, sl, :] += y[-od:]


def kernel(x, W_cls, b_cls, W_reg, b_reg, W_dir, b_dir):
    B, C, H, W = x.shape
    Oc, Or, Od = W_cls.shape[0], W_reg.shape[0], W_dir.shape[0]
    Ot = Oc + Or + Od
    HB = 8
    CB = 32
    n_g = C // CB
    assert H % HB == 0 and C % CB == 0

    w_all = jnp.concatenate([W_cls, W_reg, W_dir], axis=0)
    b_all = jnp.concatenate([b_cls, b_reg, b_dir], axis=0)[:, None]
    x3 = x.reshape(B * C, H, W)

    body = functools.partial(_heads_kernel, Oc, Od, HB)
    outs = pl.pallas_call(
        body,
        grid=(B, n_g),
        in_specs=[
            pl.BlockSpec((Ot, C), lambda b, g: (0, 0)),
            pl.BlockSpec((Ot, 1), lambda b, g: (0, 0)),
            pl.BlockSpec((CB, H, W), lambda b, g: (b * n_g + g, 0, 0)),
        ],
        out_specs=[
            pl.BlockSpec((1, Oc, H, W), lambda b, g: (b, 0, 0, 0)),
            pl.BlockSpec((1, Or, H, W), lambda b, g: (b, 0, 0, 0)),
            pl.BlockSpec((1, Od, H, W), lambda b, g: (b, 0, 0, 0)),
        ],
        out_shape=[
            jax.ShapeDtypeStruct((B, Oc, H, W), jnp.float32),
            jax.ShapeDtypeStruct((B, Or, H, W), jnp.float32),
            jax.ShapeDtypeStruct((B, Od, H, W), jnp.float32),
        ],
        scratch_shapes=[
            pltpu.VMEM((n_g, Ot * HB, CB * HB), jnp.float32),
        ],
        compiler_params=pltpu.CompilerParams(
            dimension_semantics=("parallel", "parallel")),
    )(w_all, b_all, x3)
    return tuple(outs)
